# baseline (speedup 1.0000x reference)
import functools

import jax
import jax.numpy as jnp
from jax import lax
from jax.experimental import pallas as pl
from jax.experimental.pallas import tpu as pltpu
from jax.experimental.pallas import tpu_sc as plsc

BATCH = 16384
FEAT = 128
OUT_DIM = 1

NC = 2
NS = 16
L = 16
NW = NC * NS
BPW = BATCH // NW
CHUNK = 128
NCHUNK = BPW // CHUNK

_sc_mesh = plsc.VectorSubcoreMesh(core_axis_name="c", subcore_axis_name="s")


@functools.partial(
    pl.kernel,
    mesh=_sc_mesh,
    out_type=jax.ShapeDtypeStruct((NW, NCHUNK, CHUNK), jnp.float32),
    scratch_types=[
        pltpu.VMEM((NCHUNK, CHUNK), jnp.int32),
        pltpu.VMEM((NCHUNK, CHUNK), jnp.int32),
        pltpu.VMEM((NCHUNK, CHUNK), jnp.float32),
        pltpu.VMEM((NCHUNK, CHUNK), jnp.float32),
        pltpu.SemaphoreType.DMA,
        pltpu.SemaphoreType.DMA,
    ],
)
def _sc_gather_sum(user_hbm, item_hbm, etab_u, etab_i, out_hbm,
                   uidx, iidx, uval, ival, sem_u, sem_i):
    wid = lax.axis_index("s") * NC + lax.axis_index("c")
    pltpu.sync_copy(user_hbm.at[wid], uidx)
    pltpu.sync_copy(item_hbm.at[wid], iidx)
    copies = []
    for j in range(NCHUNK):
        copies.append(pltpu.async_copy(etab_u.at[uidx.at[j]], uval.at[j], sem_u))
        copies.append(pltpu.async_copy(etab_i.at[iidx.at[j]], ival.at[j], sem_i))
    for c in copies:
        c.wait()
    for j in range(NCHUNK):
        for i in range(CHUNK // L):
            sl = pl.ds(i * L, L)
            uval[j, sl] = uval[j, sl] + ival[j, sl]
    pltpu.sync_copy(uval, out_hbm.at[wid])


def _tc_body(g_ref, uf_ref, if_ref, wu_ref, wi_ref, c_ref, out_ref):
    acc = jnp.dot(uf_ref[...], wu_ref[...], preferred_element_type=jnp.float32)
    acc = acc + jnp.dot(if_ref[...], wi_ref[...], preferred_element_type=jnp.float32)
    out_ref[...] = acc + g_ref[...] + c_ref[0]


BB = 2048


def _dense_combine(g, uf, itf, wu, wi, c):
    return pl.pallas_call(
        _tc_body,
        grid=(BATCH // BB,),
        in_specs=[
            pl.BlockSpec((BB, 1), lambda i: (i, 0)),
            pl.BlockSpec((BB, FEAT), lambda i: (i, 0)),
            pl.BlockSpec((BB, FEAT), lambda i: (i, 0)),
            pl.BlockSpec((FEAT, OUT_DIM), lambda i: (0, 0)),
            pl.BlockSpec((FEAT, OUT_DIM), lambda i: (0, 0)),
            pl.BlockSpec(memory_space=pltpu.SMEM),
        ],
        out_specs=pl.BlockSpec((BB, 1), lambda i: (i, 0)),
        out_shape=jax.ShapeDtypeStruct((BATCH, 1), jnp.float32),
    )(g, uf, itf, wu, wi, c)


def kernel(users_features, items_features, user, item, W_user, b_user,
           W_item, b_item, embed_user, embed_item, bias):
    etab_u = embed_user.reshape(-1)
    etab_i = embed_item.reshape(-1)
    u_idx = user.astype(jnp.int32).reshape(NW, NCHUNK, CHUNK)
    i_idx = item.astype(jnp.int32).reshape(NW, NCHUNK, CHUNK)
    g = _sc_gather_sum(u_idx, i_idx, etab_u, etab_i)
    c = (b_user + b_item + bias).astype(jnp.float32)
    out = _dense_combine(g.reshape(BATCH, 1), users_features, items_features,
                         W_user, W_item, c)
    return out


# speedup vs baseline: 2.7809x; 2.7809x over previous
import functools

import jax
import jax.numpy as jnp
from jax import lax
from jax.experimental import pallas as pl
from jax.experimental.pallas import tpu as pltpu
from jax.experimental.pallas import tpu_sc as plsc

BATCH = 16384
FEAT = 128
OUT_DIM = 1
TAB = 1000000

NC = 2
NS = 16
L = 16
NW = NC * NS
BPW = BATCH // NW
CHUNK = 128
NCHUNK = BPW // CHUNK

_sc_mesh = plsc.VectorSubcoreMesh(core_axis_name="c", subcore_axis_name="s")


@functools.partial(
    pl.kernel,
    mesh=_sc_mesh,
    out_type=jax.ShapeDtypeStruct((NW, NCHUNK, CHUNK), jnp.float32),
    scratch_types=[
        pltpu.VMEM((NCHUNK, CHUNK), jnp.int32),
        pltpu.VMEM((NCHUNK, CHUNK), jnp.int32),
        pltpu.VMEM((NCHUNK, CHUNK), jnp.float32),
        pltpu.VMEM((NCHUNK, CHUNK), jnp.float32),
        pltpu.SemaphoreType.DMA,
        pltpu.SemaphoreType.DMA,
    ],
)
def _sc_gather_sum(user_hbm, item_hbm, etab_u, etab_i, out_hbm,
                   uidx, iidx, uval, ival, sem_u, sem_i):
    wid = lax.axis_index("s") * NC + lax.axis_index("c")
    pltpu.sync_copy(user_hbm.at[wid], uidx)
    pltpu.sync_copy(item_hbm.at[wid], iidx)
    tab_u = etab_u.at[0]
    tab_i = etab_i.at[0]
    copies = []
    for j in range(NCHUNK):
        copies.append(pltpu.async_copy(tab_u.at[uidx.at[j]], uval.at[j], sem_u))
        copies.append(pltpu.async_copy(tab_i.at[iidx.at[j]], ival.at[j], sem_i))
    for c in copies:
        c.wait()
    for j in range(NCHUNK):
        for i in range(CHUNK // L):
            sl = pl.ds(i * L, L)
            uval[j, sl] = uval[j, sl] + ival[j, sl]
    pltpu.sync_copy(uval, out_hbm.at[wid])


def _tc_body(g_ref, uf_ref, if_ref, wu_ref, wi_ref, c_ref, out_ref):
    acc = jnp.dot(uf_ref[...], wu_ref[...], preferred_element_type=jnp.float32)
    acc = acc + jnp.dot(if_ref[...], wi_ref[...], preferred_element_type=jnp.float32)
    out_ref[...] = acc + g_ref[...] + c_ref[0]


BB = 2048


def _dense_combine(g, uf, itf, wu, wi, c):
    return pl.pallas_call(
        _tc_body,
        grid=(BATCH // BB,),
        in_specs=[
            pl.BlockSpec((BB, 1), lambda i: (i, 0)),
            pl.BlockSpec((BB, FEAT), lambda i: (i, 0)),
            pl.BlockSpec((BB, FEAT), lambda i: (i, 0)),
            pl.BlockSpec((FEAT, OUT_DIM), lambda i: (0, 0)),
            pl.BlockSpec((FEAT, OUT_DIM), lambda i: (0, 0)),
            pl.BlockSpec(memory_space=pltpu.SMEM),
        ],
        out_specs=pl.BlockSpec((BB, 1), lambda i: (i, 0)),
        out_shape=jax.ShapeDtypeStruct((BATCH, 1), jnp.float32),
    )(g, uf, itf, wu, wi, c)


def kernel(users_features, items_features, user, item, W_user, b_user,
           W_item, b_item, embed_user, embed_item, bias):
    u_idx = user.astype(jnp.int32).reshape(NW, NCHUNK, CHUNK)
    i_idx = item.astype(jnp.int32).reshape(NW, NCHUNK, CHUNK)
    g = _sc_gather_sum(u_idx, i_idx, embed_user.reshape(1, TAB),
                       embed_item.reshape(1, TAB))
    c = (b_user + b_item + bias).astype(jnp.float32)
    return _dense_combine(g.reshape(BATCH, 1), users_features, items_features,
                          W_user, W_item, c)


# speedup vs baseline: 3.6345x; 1.3069x over previous
import functools

import jax
import jax.numpy as jnp
from jax import lax
from jax.experimental import pallas as pl
from jax.experimental.pallas import tpu as pltpu
from jax.experimental.pallas import tpu_sc as plsc

BATCH = 16384
FEAT = 128
OUT_DIM = 1
TAB = 1000000

NC = 2
NS = 16
L = 16
NW = NC * NS
BPW = BATCH // NW
CHUNK = 128
NCHUNK = BPW // CHUNK
ROWS = BATCH // CHUNK
RPW = ROWS // NW

_sc_mesh = plsc.VectorSubcoreMesh(core_axis_name="c", subcore_axis_name="s")


@functools.partial(
    pl.kernel,
    mesh=_sc_mesh,
    out_type=jax.ShapeDtypeStruct((1, BATCH), jnp.float32),
    scratch_types=[
        pltpu.VMEM((NCHUNK, CHUNK), jnp.int32),
        pltpu.VMEM((NCHUNK, CHUNK), jnp.int32),
        pltpu.VMEM((NCHUNK, CHUNK), jnp.float32),
        pltpu.VMEM((NCHUNK, CHUNK), jnp.float32),
        pltpu.VMEM((RPW, CHUNK), jnp.float32),
        pltpu.VMEM((BPW,), jnp.float32),
        pltpu.VMEM((L,), jnp.float32),
        pltpu.SemaphoreType.DMA,
        pltpu.SemaphoreType.DMA,
    ],
)
def _sc_gather_combine(user_hbm, item_hbm, etab_u, etab_i, dense_hbm, c_hbm,
                       out_hbm, uidx, iidx, uval, ival, dval, sval, cval,
                       sem_u, sem_i):
    wid = lax.axis_index("s") * NC + lax.axis_index("c")
    base = wid * BPW
    pltpu.sync_copy(user_hbm.at[wid], uidx)
    pltpu.sync_copy(item_hbm.at[wid], iidx)
    pltpu.sync_copy(dense_hbm.at[pl.ds(wid * RPW, RPW)], dval)
    pltpu.sync_copy(c_hbm, cval)
    tab_u = etab_u.at[0]
    tab_i = etab_i.at[0]
    copies = []
    for j in range(NCHUNK):
        copies.append(pltpu.async_copy(tab_u.at[uidx.at[j]], uval.at[j], sem_u))
        copies.append(pltpu.async_copy(tab_i.at[iidx.at[j]], ival.at[j], sem_i))
    for c in copies:
        c.wait()
    c_vec = cval[...]
    for j in range(NCHUNK):
        for i in range(CHUNK // L):
            sl = pl.ds(i * L, L)
            sval[pl.ds(j * CHUNK + i * L, L)] = (
                uval[j, sl] + ival[j, sl] + dval[j, sl] + c_vec)
    pltpu.sync_copy(sval, out_hbm.at[0].at[pl.ds(base, BPW)])


def _tc_body(uf_ref, if_ref, wu_ref, wi_ref, out_ref):
    acc = jnp.dot(uf_ref[...], wu_ref[...], preferred_element_type=jnp.float32)
    acc = acc + jnp.dot(if_ref[...], wi_ref[...], preferred_element_type=jnp.float32)
    out_ref[...] = acc.reshape(out_ref.shape)


BB = 2048
RB = BB // CHUNK


def _dense(uf, itf, wu, wi):
    return pl.pallas_call(
        _tc_body,
        grid=(BATCH // BB,),
        in_specs=[
            pl.BlockSpec((BB, FEAT), lambda i: (i, 0)),
            pl.BlockSpec((BB, FEAT), lambda i: (i, 0)),
            pl.BlockSpec((FEAT, OUT_DIM), lambda i: (0, 0)),
            pl.BlockSpec((FEAT, OUT_DIM), lambda i: (0, 0)),
        ],
        out_specs=pl.BlockSpec((RB, CHUNK), lambda i: (i, 0)),
        out_shape=jax.ShapeDtypeStruct((ROWS, CHUNK), jnp.float32),
    )(uf, itf, wu, wi)


def kernel(users_features, items_features, user, item, W_user, b_user,
           W_item, b_item, embed_user, embed_item, bias):
    u_idx = user.astype(jnp.int32).reshape(NW, NCHUNK, CHUNK)
    i_idx = item.astype(jnp.int32).reshape(NW, NCHUNK, CHUNK)
    dense = _dense(users_features, items_features, W_user, W_item)
    c16 = jnp.broadcast_to((b_user + b_item + bias).astype(jnp.float32), (L,))
    out_row = _sc_gather_combine(u_idx, i_idx, embed_user.reshape(1, TAB),
                                 embed_item.reshape(1, TAB), dense, c16)
    return out_row.reshape(BATCH, OUT_DIM)


# speedup vs baseline: 4.2780x; 1.1771x over previous
import functools

import jax
import jax.numpy as jnp
from jax import lax
from jax.experimental import pallas as pl
from jax.experimental.pallas import tpu as pltpu
from jax.experimental.pallas import tpu_sc as plsc

BATCH = 16384
FEAT = 128
OUT_DIM = 1
TAB = 1000000

NC = 2
NS = 16
L = 16
NW = NC * NS
BPW = BATCH // NW
CHUNK = 128
NCHUNK = BPW // CHUNK
ROWS = BATCH // CHUNK
RPW = ROWS // NW

_sc_mesh = plsc.VectorSubcoreMesh(core_axis_name="c", subcore_axis_name="s")


@functools.partial(
    pl.kernel,
    mesh=_sc_mesh,
    out_type=jax.ShapeDtypeStruct((ROWS, CHUNK), jnp.float32),
    scratch_types=[
        pltpu.VMEM((NCHUNK, CHUNK), jnp.int32),
        pltpu.VMEM((NCHUNK, CHUNK), jnp.int32),
        pltpu.VMEM((NCHUNK, CHUNK), jnp.float32),
        pltpu.VMEM((NCHUNK, CHUNK), jnp.float32),
        pltpu.SemaphoreType.DMA,
        pltpu.SemaphoreType.DMA,
    ],
)
def _sc_gather_sum(user_hbm, item_hbm, etab_u, etab_i, out_hbm,
                   uidx, iidx, uval, ival, sem_u, sem_i):
    wid = lax.axis_index("s") * NC + lax.axis_index("c")
    pltpu.sync_copy(user_hbm.at[wid], uidx)
    pltpu.sync_copy(item_hbm.at[wid], iidx)
    tab_u = etab_u.at[0]
    tab_i = etab_i.at[0]
    copies = []
    for j in range(NCHUNK):
        copies.append(pltpu.async_copy(tab_u.at[uidx.at[j]], uval.at[j], sem_u))
        copies.append(pltpu.async_copy(tab_i.at[iidx.at[j]], ival.at[j], sem_i))
    for c in copies:
        c.wait()
    for j in range(NCHUNK):
        for i in range(CHUNK // L):
            sl = pl.ds(i * L, L)
            uval[j, sl] = uval[j, sl] + ival[j, sl]
    pltpu.sync_copy(uval, out_hbm.at[pl.ds(wid * RPW, RPW)])


def _tc_dense_body(uf_ref, if_ref, wu_ref, wi_ref, out_ref):
    acc = jnp.dot(uf_ref[...], wu_ref[...], preferred_element_type=jnp.float32)
    acc = acc + jnp.dot(if_ref[...], wi_ref[...], preferred_element_type=jnp.float32)
    out_ref[...] = acc.reshape(out_ref.shape)


BB = 2048
RB = BB // CHUNK


def _dense(uf, itf, wu, wi):
    return pl.pallas_call(
        _tc_dense_body,
        grid=(BATCH // BB,),
        in_specs=[
            pl.BlockSpec((BB, FEAT), lambda i: (i, 0)),
            pl.BlockSpec((BB, FEAT), lambda i: (i, 0)),
            pl.BlockSpec((FEAT, OUT_DIM), lambda i: (0, 0)),
            pl.BlockSpec((FEAT, OUT_DIM), lambda i: (0, 0)),
        ],
        out_specs=pl.BlockSpec((RB, CHUNK), lambda i: (i, 0)),
        out_shape=jax.ShapeDtypeStruct((ROWS, CHUNK), jnp.float32),
    )(uf, itf, wu, wi)


def _tc_combine_body(d_ref, g_ref, c_ref, out_ref):
    out_ref[...] = d_ref[...] + g_ref[...] + c_ref[0]


def _combine(dense, g, c):
    return pl.pallas_call(
        _tc_combine_body,
        in_specs=[
            pl.BlockSpec((ROWS, CHUNK), lambda: (0, 0)),
            pl.BlockSpec((ROWS, CHUNK), lambda: (0, 0)),
            pl.BlockSpec(memory_space=pltpu.SMEM),
        ],
        out_specs=pl.BlockSpec((ROWS, CHUNK), lambda: (0, 0)),
        out_shape=jax.ShapeDtypeStruct((ROWS, CHUNK), jnp.float32),
    )(dense, g, c)


def kernel(users_features, items_features, user, item, W_user, b_user,
           W_item, b_item, embed_user, embed_item, bias):
    u_idx = user.astype(jnp.int32).reshape(NW, NCHUNK, CHUNK)
    i_idx = item.astype(jnp.int32).reshape(NW, NCHUNK, CHUNK)
    g = _sc_gather_sum(u_idx, i_idx, embed_user.reshape(1, TAB),
                       embed_item.reshape(1, TAB))
    dense = _dense(users_features, items_features, W_user, W_item)
    c = (b_user + b_item + bias).astype(jnp.float32)
    out2d = _combine(dense, g, c)
    return out2d.reshape(BATCH, OUT_DIM)


# speedup vs baseline: 4.5406x; 1.0614x over previous
import functools

import jax
import jax.numpy as jnp
from jax import lax
from jax.experimental import pallas as pl
from jax.experimental.pallas import tpu as pltpu
from jax.experimental.pallas import tpu_sc as plsc

BATCH = 16384
FEAT = 128
OUT_DIM = 1
TAB = 1000000

NC = 2
NS = 16
L = 16
NW = NC * NS
BPW = BATCH // NW
CHUNK = 128
NCHUNK = BPW // CHUNK
ROWS = BATCH // CHUNK
RPW = ROWS // NW

_sc_mesh = plsc.VectorSubcoreMesh(core_axis_name="c", subcore_axis_name="s")


@functools.partial(
    pl.kernel,
    mesh=_sc_mesh,
    out_type=jax.ShapeDtypeStruct((ROWS, CHUNK), jnp.float32),
    scratch_types=[
        pltpu.VMEM((NCHUNK, CHUNK), jnp.int32),
        pltpu.VMEM((NCHUNK, CHUNK), jnp.int32),
        pltpu.VMEM((NCHUNK, CHUNK), jnp.float32),
        pltpu.VMEM((NCHUNK, CHUNK), jnp.float32),
        pltpu.SemaphoreType.DMA,
        pltpu.SemaphoreType.DMA,
    ],
)
def _sc_gather_sum(user_hbm, item_hbm, etab_u, etab_i, out_hbm,
                   uidx, iidx, uval, ival, sem_u, sem_i):
    wid = lax.axis_index("s") * NC + lax.axis_index("c")
    pltpu.sync_copy(user_hbm.at[wid], uidx)
    pltpu.sync_copy(item_hbm.at[wid], iidx)
    tab_u = etab_u.at[0]
    tab_i = etab_i.at[0]
    copies = []
    for j in range(NCHUNK):
        copies.append(pltpu.async_copy(tab_u.at[uidx.at[j]], uval.at[j], sem_u))
        copies.append(pltpu.async_copy(tab_i.at[iidx.at[j]], ival.at[j], sem_i))
    for c in copies:
        c.wait()
    for j in range(NCHUNK):
        for i in range(CHUNK // L):
            sl = pl.ds(i * L, L)
            uval[j, sl] = uval[j, sl] + ival[j, sl]
    pltpu.sync_copy(uval, out_hbm.at[pl.ds(wid * RPW, RPW)])


def _tc_dense_body(uf_ref, if_ref, wu_ref, wi_ref, out_ref):
    acc = jnp.dot(uf_ref[...], wu_ref[...], preferred_element_type=jnp.float32)
    acc = acc + jnp.dot(if_ref[...], wi_ref[...], preferred_element_type=jnp.float32)
    out_ref[...] = acc.reshape(out_ref.shape)


BB = 4096
RB = BB // CHUNK


def _dense(uf, itf, wu, wi):
    return pl.pallas_call(
        _tc_dense_body,
        grid=(BATCH // BB,),
        in_specs=[
            pl.BlockSpec((BB, FEAT), lambda i: (i, 0)),
            pl.BlockSpec((BB, FEAT), lambda i: (i, 0)),
            pl.BlockSpec((FEAT, OUT_DIM), lambda i: (0, 0)),
            pl.BlockSpec((FEAT, OUT_DIM), lambda i: (0, 0)),
        ],
        out_specs=pl.BlockSpec((RB, CHUNK), lambda i: (i, 0)),
        out_shape=jax.ShapeDtypeStruct((ROWS, CHUNK), jnp.float32),
    )(uf, itf, wu, wi)


def _tc_combine_body(d_ref, g_ref, c_ref, out_ref):
    out_ref[...] = d_ref[...] + g_ref[...] + c_ref[0]


def _combine(dense, g, c):
    return pl.pallas_call(
        _tc_combine_body,
        in_specs=[
            pl.BlockSpec((ROWS, CHUNK), lambda: (0, 0)),
            pl.BlockSpec((ROWS, CHUNK), lambda: (0, 0)),
            pl.BlockSpec(memory_space=pltpu.SMEM),
        ],
        out_specs=pl.BlockSpec((ROWS, CHUNK), lambda: (0, 0)),
        out_shape=jax.ShapeDtypeStruct((ROWS, CHUNK), jnp.float32),
    )(dense, g, c)


def kernel(users_features, items_features, user, item, W_user, b_user,
           W_item, b_item, embed_user, embed_item, bias):
    u_idx = user.astype(jnp.int32).reshape(NW, NCHUNK, CHUNK)
    i_idx = item.astype(jnp.int32).reshape(NW, NCHUNK, CHUNK)
    g = _sc_gather_sum(u_idx, i_idx, embed_user.reshape(1, TAB),
                       embed_item.reshape(1, TAB))
    dense = _dense(users_features, items_features, W_user, W_item)
    c = (b_user + b_item + bias).astype(jnp.float32)
    out2d = _combine(dense, g, c)
    return out2d.reshape(BATCH, OUT_DIM)


# speedup vs baseline: 4.5791x; 1.0085x over previous
import functools

import jax
import jax.numpy as jnp
from jax import lax
from jax.experimental import pallas as pl
from jax.experimental.pallas import tpu as pltpu
from jax.experimental.pallas import tpu_sc as plsc

BATCH = 16384
FEAT = 128
OUT_DIM = 1
TAB = 1000000

NC = 2
NS = 16
L = 16
NW = NC * NS
BPW = BATCH // NW
CHUNK = 128
NCHUNK = BPW // CHUNK
ROWS = BATCH // CHUNK
RPW = ROWS // NW

_sc_mesh = plsc.VectorSubcoreMesh(core_axis_name="c", subcore_axis_name="s")


@functools.partial(
    pl.kernel,
    mesh=_sc_mesh,
    out_type=jax.ShapeDtypeStruct((ROWS, CHUNK), jnp.float32),
    scratch_types=[
        pltpu.VMEM((NCHUNK, CHUNK), jnp.int32),
        pltpu.VMEM((NCHUNK, CHUNK), jnp.int32),
        pltpu.VMEM((NCHUNK, CHUNK), jnp.float32),
        pltpu.VMEM((NCHUNK, CHUNK), jnp.float32),
        pltpu.SemaphoreType.DMA,
        pltpu.SemaphoreType.DMA,
    ],
)
def _sc_gather_sum(user_hbm, item_hbm, etab_u, etab_i, out_hbm,
                   uidx, iidx, uval, ival, sem_u, sem_i):
    wid = lax.axis_index("s") * NC + lax.axis_index("c")
    pltpu.sync_copy(user_hbm.at[wid], uidx)
    pltpu.sync_copy(item_hbm.at[wid], iidx)
    tab_u = etab_u.at[0]
    tab_i = etab_i.at[0]
    copies = []
    for j in range(NCHUNK):
        copies.append(pltpu.async_copy(tab_u.at[uidx.at[j]], uval.at[j], sem_u))
        copies.append(pltpu.async_copy(tab_i.at[iidx.at[j]], ival.at[j], sem_i))
    for c in copies:
        c.wait()
    for j in range(NCHUNK):
        for i in range(CHUNK // L):
            sl = pl.ds(i * L, L)
            uval[j, sl] = uval[j, sl] + ival[j, sl]
    pltpu.sync_copy(uval, out_hbm.at[pl.ds(wid * RPW, RPW)])


def _tc_dense_body(uf_ref, if_ref, wu_ref, wi_ref, out_ref):
    acc = jnp.dot(uf_ref[...], wu_ref[...], preferred_element_type=jnp.float32)
    acc = acc + jnp.dot(if_ref[...], wi_ref[...], preferred_element_type=jnp.float32)
    out_ref[...] = acc.reshape(out_ref.shape)


BB = 8192
RB = BB // CHUNK


def _dense(uf, itf, wu, wi):
    return pl.pallas_call(
        _tc_dense_body,
        grid=(BATCH // BB,),
        in_specs=[
            pl.BlockSpec((BB, FEAT), lambda i: (i, 0)),
            pl.BlockSpec((BB, FEAT), lambda i: (i, 0)),
            pl.BlockSpec((FEAT, OUT_DIM), lambda i: (0, 0)),
            pl.BlockSpec((FEAT, OUT_DIM), lambda i: (0, 0)),
        ],
        out_specs=pl.BlockSpec((RB, CHUNK), lambda i: (i, 0)),
        out_shape=jax.ShapeDtypeStruct((ROWS, CHUNK), jnp.float32),
    )(uf, itf, wu, wi)


def _tc_combine_body(d_ref, g_ref, c_ref, out_ref):
    out_ref[...] = d_ref[...] + g_ref[...] + c_ref[0]


def _combine(dense, g, c):
    return pl.pallas_call(
        _tc_combine_body,
        in_specs=[
            pl.BlockSpec((ROWS, CHUNK), lambda: (0, 0)),
            pl.BlockSpec((ROWS, CHUNK), lambda: (0, 0)),
            pl.BlockSpec(memory_space=pltpu.SMEM),
        ],
        out_specs=pl.BlockSpec((ROWS, CHUNK), lambda: (0, 0)),
        out_shape=jax.ShapeDtypeStruct((ROWS, CHUNK), jnp.float32),
    )(dense, g, c)


def kernel(users_features, items_features, user, item, W_user, b_user,
           W_item, b_item, embed_user, embed_item, bias):
    u_idx = user.astype(jnp.int32).reshape(NW, NCHUNK, CHUNK)
    i_idx = item.astype(jnp.int32).reshape(NW, NCHUNK, CHUNK)
    g = _sc_gather_sum(u_idx, i_idx, embed_user.reshape(1, TAB),
                       embed_item.reshape(1, TAB))
    dense = _dense(users_features, items_features, W_user, W_item)
    c = (b_user + b_item + bias).astype(jnp.float32)
    out2d = _combine(dense, g, c)
    return out2d.reshape(BATCH, OUT_DIM)
